# Initial kernel scaffold; baseline (speedup 1.0000x reference)
import jax
import jax.numpy as jnp
from jax.experimental import pallas as pl


def kernel(y_pred, y_true, weights):
    raise NotImplementedError("write your pallas kernel here")


# speedup vs baseline: 19.1898x; 19.1898x over previous
import jax
import jax.numpy as jnp
from jax.experimental import pallas as pl
from jax.experimental.pallas import tpu as pltpu

_B = 16384
_N = 1000
_BLK = 512
_GRID = _B // _BLK


def _pass_body(y_ref, lab_ref, out_ref, acc_ref):
    i = pl.program_id(0)

    @pl.when(i == 0)
    def _init():
        for k in range(5):
            acc_ref[k] = 0.0

    y = y_ref[...]
    e = jnp.exp(y)
    lab = lab_ref[...]
    col = jax.lax.broadcasted_iota(jnp.int32, (_BLK, _N), 1)
    mask = col == lab

    rowsum = jnp.sum(e, axis=1, keepdims=True)
    s_lse = jnp.sum(jnp.log(rowsum))
    s_e2 = jnp.sum(e * e)
    s_sp = jnp.sum(jnp.log1p(jnp.exp(e)))
    s_tval = jnp.sum(jnp.where(mask, y, 0.0))
    s_pt = jnp.sum(jnp.where(mask, e, 0.0))

    acc_ref[0] += s_e2
    acc_ref[1] += s_pt
    acc_ref[2] += s_tval
    acc_ref[3] += s_lse
    acc_ref[4] += s_sp

    @pl.when(i == _GRID - 1)
    def _fin():
        for k in range(5):
            out_ref[k] = acc_ref[k]


def kernel(y_pred, y_true, weights):
    lab = y_true.astype(jnp.int32).reshape(_B, 1)
    sums = pl.pallas_call(
        _pass_body,
        grid=(_GRID,),
        in_specs=[
            pl.BlockSpec((_BLK, _N), lambda i: (i, i * 0)),
            pl.BlockSpec((_BLK, 1), lambda i: (i, i * 0)),
        ],
        out_specs=pl.BlockSpec((5,), lambda i: (i * 0,), memory_space=pltpu.SMEM),
        out_shape=jax.ShapeDtypeStruct((5,), jnp.float32),
        scratch_shapes=[pltpu.SMEM((5,), jnp.float32)],
    )(y_pred, lab)
    s_e2 = sums[0].astype(jnp.float64)
    s_pt = sums[1].astype(jnp.float64)
    s_tval = sums[2].astype(jnp.float64)
    s_lse = sums[3].astype(jnp.float64)
    s_sp = sums[4].astype(jnp.float64)

    w = jax.nn.softplus(weights)
    bn = float(_B * _N)
    mse = (s_e2 - 2.0 * s_pt + float(_B)) / bn
    ce = (s_lse - s_tval) / float(_B)
    mlsm = (s_sp - s_pt) / bn
    return w[0] * mse + w[1] * ce + w[2] * mlsm
